# baseline (speedup 1.0000x reference)
import math
import functools

import jax
import jax.numpy as jnp
from jax import lax
from jax.experimental import pallas as pl
from jax.experimental.pallas import tpu as pltpu
from jax.experimental.pallas import tpu_sc as plsc


def _logits_body(x_ref, gw_ref, out_ref):
    out_ref[...] = lax.dot_general(
        gw_ref[...], x_ref[...], (((1,), (1,)), ((), ())),
        preferred_element_type=jnp.float32,
        precision=lax.Precision.HIGHEST)


def _router_logits(xf, gate_w):
    T, H = xf.shape
    E = gate_w.shape[0]
    return pl.pallas_call(
        _logits_body,
        out_shape=jax.ShapeDtypeStruct((E, T), jnp.float32),
    )(xf, gate_w)


def _sc_select(scores, cap):
    E, T = scores.shape
    info = plsc.get_sparse_core_info()
    nc = info.num_cores
    nv = T // 16
    mesh = plsc.VectorSubcoreMesh(core_axis_name="c", subcore_axis_name="s")

    def body(scores_hbm, sel_hbm, ss_hbm, scores_v, keys_v, sel_v, ss_v):
        wid = lax.axis_index("s") * nc + lax.axis_index("c")

        @pl.when(wid < E)
        def _work():
            pltpu.sync_copy(scores_hbm.at[wid], scores_v)

            magic = jnp.full((16,), 0x7FFFFFFF, jnp.int32)
            thirtyone = jnp.full((16,), 31, jnp.int32)

            def _scal(v):
                return lax.squeeze(lax.slice(v, (0,), (1,)), (0,))

            def _pc(m):
                return plsc.all_reduce_population_count(m)

            def _mk_keys(i, c):
                s = scores_v[pl.ds(i * 16, 16)]
                b = lax.bitcast_convert_type(s, jnp.int32)
                keys_v[pl.ds(i * 16, 16)] = b ^ (
                    lax.shift_right_arithmetic(b, thirtyone) & magic)
                return c
            lax.fori_loop(0, nv, _mk_keys, 0)

            def _count(thr, strict):
                tv = jnp.full((16,), thr, jnp.int32)

                def _acc(i, a):
                    k = keys_v[pl.ds(i * 16, 16)]
                    m = (k > tv) if strict else (k >= tv)
                    return a + _pc(m)
                cv = lax.fori_loop(0, nv, _acc,
                                   jnp.zeros((16,), jnp.int32))
                return _scal(cv)

            def _bis(_, carry):
                lo, hi = carry
                mid = ((lo >> 1) + (hi >> 1) + (lo & hi & 1)
                       + ((lo ^ hi) & 1))
                ge = _count(mid, False) >= cap
                return (jnp.where(ge, mid, lo),
                        jnp.where(ge, hi, mid - 1))
            thr, _ = lax.fori_loop(
                0, 32, _bis,
                (jnp.int32(-(2 ** 31)), jnp.int32(2 ** 31 - 1)))

            ties_needed = cap - _count(thr, True)
            tv = jnp.full((16,), thr, jnp.int32)

            def _count_tie(ti):
                tiv = jnp.full((16,), ti, jnp.int32)

                def _acc(i, a):
                    k = keys_v[pl.ds(i * 16, 16)]
                    tok = lax.iota(jnp.int32, 16) + jnp.full(
                        (16,), i * 16, jnp.int32)
                    m = jnp.logical_and(k == tv, tok <= tiv)
                    return a + _pc(m)
                cv = lax.fori_loop(0, nv, _acc,
                                   jnp.zeros((16,), jnp.int32))
                return _scal(cv)

            def _bis2(_, carry):
                lo, hi = carry
                mid = (lo >> 1) + (hi >> 1) + (lo & hi & 1)
                ge = _count_tie(mid) >= ties_needed
                return (jnp.where(ge, lo, mid + 1),
                        jnp.where(ge, mid, hi))
            _, tie_hi = lax.fori_loop(
                0, 13, _bis2, (jnp.int32(0), jnp.int32(T - 1)))
            tie_v = jnp.full((16,), tie_hi, jnp.int32)

            def _compact(i, pos):
                k = keys_v[pl.ds(i * 16, 16)]
                s = scores_v[pl.ds(i * 16, 16)]
                tok = lax.iota(jnp.int32, 16) + jnp.full((16,), i * 16,
                                                         jnp.int32)
                m = jnp.logical_or(
                    k > tv,
                    jnp.logical_and(k == tv, tok <= tie_v))
                plsc.store_compressed(sel_v.at[pl.ds(pos, 16)], tok, mask=m)
                plsc.store_compressed(ss_v.at[pl.ds(pos, 16)], s, mask=m)
                return pos + _scal(_pc(m))
            lax.fori_loop(0, nv, _compact, jnp.int32(0))

            pltpu.sync_copy(sel_v.at[pl.ds(0, cap)], sel_hbm.at[wid])
            pltpu.sync_copy(ss_v.at[pl.ds(0, cap)], ss_hbm.at[wid])

    k = pl.kernel(
        body, mesh=mesh,
        compiler_params=pltpu.CompilerParams(needs_layout_passes=False),
        out_type=(jax.ShapeDtypeStruct((E, cap), jnp.int32),
                  jax.ShapeDtypeStruct((E, cap), jnp.float32)),
        scratch_types=[
            pltpu.VMEM((T,), jnp.float32),
            pltpu.VMEM((T,), jnp.int32),
            pltpu.VMEM((cap + 16,), jnp.int32),
            pltpu.VMEM((cap + 16,), jnp.float32),
        ],
    )
    return k(scores)


def _ffn_body(nit, score_ref, xg_ref, w1_ref, w3_ref, w2_ref, out_ref,
              acc_ref):
    it = pl.program_id(1)

    @pl.when(it == 0)
    def _init():
        acc_ref[...] = jnp.zeros_like(acc_ref)

    xb = xg_ref[0]
    w1 = w1_ref[0].astype(jnp.bfloat16)
    w3 = w3_ref[0].astype(jnp.bfloat16)
    w2 = w2_ref[0].astype(jnp.bfloat16)
    a = lax.dot_general(xb, w1, (((1,), (1,)), ((), ())),
                        preferred_element_type=jnp.float32)
    b = lax.dot_general(xb, w3, (((1,), (1,)), ((), ())),
                        preferred_element_type=jnp.float32)
    h = (a * jax.nn.sigmoid(a) * b).astype(jnp.bfloat16)
    acc_ref[...] += lax.dot_general(h, w2, (((1,), (1,)), ((), ())),
                                    preferred_element_type=jnp.float32)

    @pl.when(it == nit - 1)
    def _fin():
        s = score_ref[0]
        m = jnp.max(s, axis=-1, keepdims=True)
        ex = jnp.exp(s - m)
        w = ex / jnp.sum(ex, axis=-1, keepdims=True)
        out_ref[0] = (acc_ref[...] * w.reshape(-1, 1)).astype(jnp.bfloat16)


def _ffn(selscore, xg, w1b, w3b, w2b):
    E, CAP, H = xg.shape
    I = w1b.shape[1]
    TI = min(512, I)
    NIT = I // TI
    grid = (E, NIT)
    return pl.pallas_call(
        functools.partial(_ffn_body, NIT),
        grid=grid,
        in_specs=[
            pl.BlockSpec((1, 1, CAP), lambda e, i: (e, 0, 0)),
            pl.BlockSpec((1, CAP, H), lambda e, i: (e, 0, 0)),
            pl.BlockSpec((1, TI, H), lambda e, i: (e, i, 0)),
            pl.BlockSpec((1, TI, H), lambda e, i: (e, i, 0)),
            pl.BlockSpec((1, H, TI), lambda e, i: (e, 0, i)),
        ],
        out_specs=pl.BlockSpec((1, CAP, H), lambda e, i: (e, 0, 0)),
        out_shape=jax.ShapeDtypeStruct((E, CAP, H), jnp.bfloat16),
        scratch_shapes=[pltpu.VMEM((CAP, H), jnp.float32)],
    )(selscore.reshape(E, 1, CAP), xg, w1b, w3b, w2b)


def _combine_body(nexp, sel_ref, eo_ref, out_ref):
    e = pl.program_id(0)
    T = out_ref.shape[0]
    CAP = sel_ref.shape[2]

    @pl.when(e == 0)
    def _init():
        out_ref[...] = jnp.zeros_like(out_ref)

    sel = sel_ref[0]
    tcol = lax.broadcasted_iota(jnp.int32, (T, CAP), 0)
    P = (tcol == sel).astype(jnp.bfloat16)
    out_ref[...] += lax.dot_general(
        P, eo_ref[0], (((1,), (0,)), ((), ())),
        preferred_element_type=jnp.float32)


def _combine(sel, eo_bf, T):
    E, CAP, H = eo_bf.shape
    return pl.pallas_call(
        functools.partial(_combine_body, E),
        grid=(E,),
        in_specs=[
            pl.BlockSpec((1, 1, CAP), lambda e: (e, 0, 0)),
            pl.BlockSpec((1, CAP, H), lambda e: (e, 0, 0)),
        ],
        out_specs=pl.BlockSpec((T, H), lambda e: (0, 0)),
        out_shape=jax.ShapeDtypeStruct((T, H), jnp.float32),
    )(sel.reshape(E, 1, CAP), eo_bf)


def kernel(x, gate_w, w1, w2, w3):
    B, S, H = x.shape
    E = gate_w.shape[0]
    T = B * S
    cap = min(int(math.ceil(T / E * 1.25)), T)

    xf = x.reshape(T, H)
    logits = (xf @ gate_w.T).T

    sel, selscore = _sc_select(logits, cap)
    xg = jnp.take(xf, sel.reshape(-1), axis=0).reshape(E, cap, H)

    eo = _ffn(selscore, xg.astype(jnp.bfloat16),
              w1, w3, w2)

    out = _combine(sel, eo, T)
    return out.reshape(B, S, H), jnp.array(0.0, dtype=jnp.float32)


# speedup vs baseline: 1.0066x; 1.0066x over previous
import math
import functools

import jax
import jax.numpy as jnp
from jax import lax
from jax.experimental import pallas as pl
from jax.experimental.pallas import tpu as pltpu
from jax.experimental.pallas import tpu_sc as plsc


def _logits_body(x_ref, gw_ref, out_ref):
    out_ref[...] = lax.dot_general(
        gw_ref[...], x_ref[...], (((1,), (1,)), ((), ())),
        preferred_element_type=jnp.float32,
        precision=lax.Precision.HIGHEST)


def _router_logits(xf, gate_w):
    T, H = xf.shape
    E = gate_w.shape[0]
    return pl.pallas_call(
        _logits_body,
        out_shape=jax.ShapeDtypeStruct((E, T), jnp.float32),
    )(xf, gate_w)


def _sc_select(scores, cap):
    E, T = scores.shape
    info = plsc.get_sparse_core_info()
    nc = info.num_cores
    nv = T // 16
    mesh = plsc.VectorSubcoreMesh(core_axis_name="c", subcore_axis_name="s")

    def body(scores_hbm, sel_hbm, ss_hbm, scores_v, keys_v, sel_v, ss_v):
        wid = lax.axis_index("s") * nc + lax.axis_index("c")

        @pl.when(wid < E)
        def _work():
            pltpu.sync_copy(scores_hbm.at[wid], scores_v)

            magic = jnp.full((16,), 0x7FFFFFFF, jnp.int32)
            thirtyone = jnp.full((16,), 31, jnp.int32)

            def _scal(v):
                return lax.squeeze(lax.slice(v, (0,), (1,)), (0,))

            def _pc(m):
                return plsc.all_reduce_population_count(m)

            def _mk_keys(i, c):
                s = scores_v[pl.ds(i * 16, 16)]
                b = lax.bitcast_convert_type(s, jnp.int32)
                keys_v[pl.ds(i * 16, 16)] = b ^ (
                    lax.shift_right_arithmetic(b, thirtyone) & magic)
                return c
            lax.fori_loop(0, nv, _mk_keys, 0)

            def _count(thr, strict):
                tv = jnp.full((16,), thr, jnp.int32)

                def _acc(i, a):
                    k = keys_v[pl.ds(i * 16, 16)]
                    m = (k > tv) if strict else (k >= tv)
                    return a + _pc(m)
                cv = lax.fori_loop(0, nv, _acc,
                                   jnp.zeros((16,), jnp.int32))
                return _scal(cv)

            def _bis(_, carry):
                lo, hi = carry
                mid = ((lo >> 1) + (hi >> 1) + (lo & hi & 1)
                       + ((lo ^ hi) & 1))
                ge = _count(mid, False) >= cap
                return (jnp.where(ge, mid, lo),
                        jnp.where(ge, hi, mid - 1))
            thr, _ = lax.fori_loop(
                0, 32, _bis,
                (jnp.int32(-(2 ** 31)), jnp.int32(2 ** 31 - 1)))

            ties_needed = cap - _count(thr, True)
            tv = jnp.full((16,), thr, jnp.int32)

            def _count_tie(ti):
                tiv = jnp.full((16,), ti, jnp.int32)

                def _acc(i, a):
                    k = keys_v[pl.ds(i * 16, 16)]
                    tok = lax.iota(jnp.int32, 16) + jnp.full(
                        (16,), i * 16, jnp.int32)
                    m = jnp.logical_and(k == tv, tok <= tiv)
                    return a + _pc(m)
                cv = lax.fori_loop(0, nv, _acc,
                                   jnp.zeros((16,), jnp.int32))
                return _scal(cv)

            def _bis2(_, carry):
                lo, hi = carry
                mid = (lo >> 1) + (hi >> 1) + (lo & hi & 1)
                ge = _count_tie(mid) >= ties_needed
                return (jnp.where(ge, lo, mid + 1),
                        jnp.where(ge, mid, hi))
            _, tie_hi = lax.fori_loop(
                0, 13, _bis2, (jnp.int32(0), jnp.int32(T - 1)))
            tie_v = jnp.full((16,), tie_hi, jnp.int32)

            def _compact(i, pos):
                k = keys_v[pl.ds(i * 16, 16)]
                s = scores_v[pl.ds(i * 16, 16)]
                tok = lax.iota(jnp.int32, 16) + jnp.full((16,), i * 16,
                                                         jnp.int32)
                m = jnp.logical_or(
                    k > tv,
                    jnp.logical_and(k == tv, tok <= tie_v))
                plsc.store_compressed(sel_v.at[pl.ds(pos, 16)], tok, mask=m)
                plsc.store_compressed(ss_v.at[pl.ds(pos, 16)], s, mask=m)
                return pos + _scal(_pc(m))
            lax.fori_loop(0, nv, _compact, jnp.int32(0))

            pltpu.sync_copy(sel_v.at[pl.ds(0, cap)], sel_hbm.at[wid])
            pltpu.sync_copy(ss_v.at[pl.ds(0, cap)], ss_hbm.at[wid])

    k = pl.kernel(
        body, mesh=mesh,
        compiler_params=pltpu.CompilerParams(needs_layout_passes=False),
        out_type=(jax.ShapeDtypeStruct((E, cap), jnp.int32),
                  jax.ShapeDtypeStruct((E, cap), jnp.float32)),
        scratch_types=[
            pltpu.VMEM((T,), jnp.float32),
            pltpu.VMEM((T,), jnp.int32),
            pltpu.VMEM((cap + 16,), jnp.int32),
            pltpu.VMEM((cap + 16,), jnp.float32),
        ],
    )
    return k(scores)


def _sc_gather(xf, sel_flat):
    T, H = xf.shape
    (NSEL,) = sel_flat.shape
    info = plsc.get_sparse_core_info()
    nc = info.num_cores
    nw = nc * info.num_subcores
    rows_pw = NSEL // nw
    chunk = 16
    nch = rows_pw // chunk
    mesh = plsc.VectorSubcoreMesh(core_axis_name="c", subcore_axis_name="s")

    def body(x_hbm, idx_hbm, out_hbm, idx_v, rows_v, sem):
        wid = lax.axis_index("s") * nc + lax.axis_index("c")
        base = wid * rows_pw
        pltpu.sync_copy(idx_hbm.at[pl.ds(base, rows_pw)], idx_v)

        def _chunk(c, carry):
            pltpu.async_copy(
                x_hbm.at[idx_v.at[pl.ds(c * chunk, chunk)]], rows_v,
                sem).wait()
            pltpu.sync_copy(rows_v,
                            out_hbm.at[pl.ds(base + c * chunk, chunk)])
            return carry
        lax.fori_loop(0, nch, _chunk, 0)

    k = pl.kernel(
        body, mesh=mesh,
        compiler_params=pltpu.CompilerParams(needs_layout_passes=False),
        out_type=jax.ShapeDtypeStruct((NSEL, H), jnp.float32),
        scratch_types=[
            pltpu.VMEM((rows_pw,), jnp.int32),
            pltpu.VMEM((chunk, H), jnp.float32),
            pltpu.SemaphoreType.DMA,
        ],
    )
    return k(xf, sel_flat)


def _ffn_body(nit, score_ref, xg_ref, w1_ref, w3_ref, w2_ref, out_ref,
              acc_ref):
    it = pl.program_id(1)

    @pl.when(it == 0)
    def _init():
        acc_ref[...] = jnp.zeros_like(acc_ref)

    xb = xg_ref[0]
    w1 = w1_ref[0].astype(jnp.bfloat16)
    w3 = w3_ref[0].astype(jnp.bfloat16)
    w2 = w2_ref[0].astype(jnp.bfloat16)
    a = lax.dot_general(xb, w1, (((1,), (1,)), ((), ())),
                        preferred_element_type=jnp.float32)
    b = lax.dot_general(xb, w3, (((1,), (1,)), ((), ())),
                        preferred_element_type=jnp.float32)
    h = (a * jax.nn.sigmoid(a) * b).astype(jnp.bfloat16)
    acc_ref[...] += lax.dot_general(h, w2, (((1,), (1,)), ((), ())),
                                    preferred_element_type=jnp.float32)

    @pl.when(it == nit - 1)
    def _fin():
        s = score_ref[0]
        m = jnp.max(s, axis=-1, keepdims=True)
        ex = jnp.exp(s - m)
        w = ex / jnp.sum(ex, axis=-1, keepdims=True)
        out_ref[0] = (acc_ref[...] * w.reshape(-1, 1)).astype(jnp.bfloat16)


def _ffn(selscore, xg, w1b, w3b, w2b):
    E, CAP, H = xg.shape
    I = w1b.shape[1]
    TI = min(512, I)
    NIT = I // TI
    grid = (E, NIT)
    return pl.pallas_call(
        functools.partial(_ffn_body, NIT),
        grid=grid,
        in_specs=[
            pl.BlockSpec((1, 1, CAP), lambda e, i: (e, 0, 0)),
            pl.BlockSpec((1, CAP, H), lambda e, i: (e, 0, 0)),
            pl.BlockSpec((1, TI, H), lambda e, i: (e, i, 0)),
            pl.BlockSpec((1, TI, H), lambda e, i: (e, i, 0)),
            pl.BlockSpec((1, H, TI), lambda e, i: (e, 0, i)),
        ],
        out_specs=pl.BlockSpec((1, CAP, H), lambda e, i: (e, 0, 0)),
        out_shape=jax.ShapeDtypeStruct((E, CAP, H), jnp.bfloat16),
        scratch_shapes=[pltpu.VMEM((CAP, H), jnp.float32)],
    )(selscore.reshape(E, 1, CAP), xg, w1b, w3b, w2b)


def _combine_body(nexp, sel_ref, eo_ref, out_ref):
    e = pl.program_id(0)
    T = out_ref.shape[0]
    CAP = sel_ref.shape[2]

    @pl.when(e == 0)
    def _init():
        out_ref[...] = jnp.zeros_like(out_ref)

    sel = sel_ref[0]
    tcol = lax.broadcasted_iota(jnp.int32, (T, CAP), 0)
    P = (tcol == sel).astype(jnp.bfloat16)
    out_ref[...] += lax.dot_general(
        P, eo_ref[0], (((1,), (0,)), ((), ())),
        preferred_element_type=jnp.float32)


def _combine(sel, eo_bf, T):
    E, CAP, H = eo_bf.shape
    return pl.pallas_call(
        functools.partial(_combine_body, E),
        grid=(E,),
        in_specs=[
            pl.BlockSpec((1, 1, CAP), lambda e: (e, 0, 0)),
            pl.BlockSpec((1, CAP, H), lambda e: (e, 0, 0)),
        ],
        out_specs=pl.BlockSpec((T, H), lambda e: (0, 0)),
        out_shape=jax.ShapeDtypeStruct((T, H), jnp.float32),
    )(sel.reshape(E, 1, CAP), eo_bf)


def kernel(x, gate_w, w1, w2, w3):
    B, S, H = x.shape
    E = gate_w.shape[0]
    T = B * S
    cap = min(int(math.ceil(T / E * 1.25)), T)

    xf = x.reshape(T, H)
    logits = (xf @ gate_w.T).T

    sel, selscore = _sc_select(logits, cap)
    xg = _sc_gather(xf, sel.reshape(-1)).reshape(E, cap, H)

    eo = _ffn(selscore, xg.astype(jnp.bfloat16),
              w1, w3, w2)

    out = _combine(sel, eo, T)
    return out.reshape(B, S, H), jnp.array(0.0, dtype=jnp.float32)
